# Initial kernel scaffold; baseline (speedup 1.0000x reference)
import jax
import jax.numpy as jnp
from jax.experimental import pallas as pl


def kernel(x, edge_index, W_qkv, b_qkv, W_ff, b_ff):
    raise NotImplementedError("write your pallas kernel here")


# speedup vs baseline: 1.0473x; 1.0473x over previous
from math import sqrt

import jax
import jax.numpy as jnp
from jax.experimental import pallas as pl

N = 10000
E = 160000
D_EMB = 256
HEADS = 8
DK = 64
DV = 64

BN = 1000


def _matmul_bias_kernel(x_ref, w_ref, b_ref, o_ref):
    o_ref[...] = (
        jnp.dot(x_ref[...], w_ref[...], preferred_element_type=jnp.float32)
        + b_ref[...]
    )


def _project(x, W_T, b):
    n, d_in = x.shape
    d_out = W_T.shape[1]
    grid = (n // BN,)
    return pl.pallas_call(
        _matmul_bias_kernel,
        grid=grid,
        in_specs=[
            pl.BlockSpec((BN, d_in), lambda i: (i, 0)),
            pl.BlockSpec((d_in, d_out), lambda i: (0, 0)),
            pl.BlockSpec((1, d_out), lambda i: (0, 0)),
        ],
        out_specs=pl.BlockSpec((BN, d_out), lambda i: (i, 0)),
        out_shape=jax.ShapeDtypeStruct((n, d_out), jnp.float32),
    )(x, W_T, b.reshape(1, d_out))


def kernel(x, edge_index, W_qkv, b_qkv, W_ff, b_ff):
    src = edge_index[0]
    dst = edge_index[1]
    qkv = _project(x, W_qkv.T, b_qkv)
    Q, K, V = jnp.split(qkv, [DK * HEADS, 2 * DK * HEADS], axis=-1)
    Q = Q.reshape(-1, HEADS, DK)
    K = K.reshape(-1, HEADS, DK)
    V = V.reshape(-1, HEADS, DV)
    scores = jnp.sum(Q[dst] * K[src], axis=-1) / sqrt(DK)
    ex = jnp.exp(scores)
    denom = jax.ops.segment_sum(ex, dst, num_segments=N)
    numer = jax.ops.segment_sum(ex[..., None] * V[src], dst, num_segments=N)
    denom = jnp.where(denom == 0.0, 1.0, denom)
    out = numer / denom[..., None]
    return _project(out.reshape(-1, HEADS * DV), W_ff.T, b_ff)


# speedup vs baseline: 6.6357x; 6.3360x over previous
import functools

import jax
import jax.numpy as jnp
from jax import lax
from jax.experimental import pallas as pl
from jax.experimental.pallas import tpu as pltpu
from jax.experimental.pallas import tpu_sc as plsc

N = 10000
E = 160000
D_EMB = 256
HEADS = 8
DK = 64
DV = 64

BN = 1000
C = 80
TILES = 16
EPT = E // TILES
CHUNKS = EPT // C
RW = 128
PSL = E * 32
NHALF = 5120
NR_ACC = 5248
ROWS_PT = NR_ACC // TILES
NACC_OUT = NHALF + NR_ACC


def _matmul_bias_kernel(x_ref, w_ref, b_ref, o_ref):
    o_ref[...] = (
        jnp.dot(x_ref[...], w_ref[...], preferred_element_type=jnp.float32)
        + b_ref[...]
    )


def _project(x, W_T, b):
    n, d_in = x.shape
    d_out = W_T.shape[1]
    return pl.pallas_call(
        _matmul_bias_kernel,
        grid=(n // BN,),
        in_specs=[
            pl.BlockSpec((BN, d_in), lambda i: (i, 0)),
            pl.BlockSpec((d_in, d_out), lambda i: (0, 0)),
            pl.BlockSpec((1, d_out), lambda i: (0, 0)),
        ],
        out_specs=pl.BlockSpec((BN, d_out), lambda i: (i, 0)),
        out_shape=jax.ShapeDtypeStruct((n, d_out), jnp.float32),
    )(x, W_T, b.reshape(1, d_out))


_SC_MESH = plsc.VectorSubcoreMesh(core_axis_name="c", subcore_axis_name="s")


@functools.partial(
    pl.kernel,
    out_type=jax.ShapeDtypeStruct((4 * PSL,), jnp.float32),
    mesh=_SC_MESH,
    scratch_types=[
        pltpu.VMEM((EPT,), jnp.int32),
        pltpu.VMEM((EPT,), jnp.int32),
        pltpu.VMEM((C,), jnp.int32),
        pltpu.VMEM((C,), jnp.int32),
        pltpu.VMEM((C, RW), jnp.float32),
        pltpu.VMEM((C, RW), jnp.float32),
        pltpu.VMEM((C * 32,), jnp.float32),
    ],
)
def _score_kernel(table, ei, slab, sall, dall, iqb, ikb, qrows, krows, prodb):
    cid = lax.axis_index("c")
    sid = lax.axis_index("s")
    ebase = sid * EPT
    pltpu.sync_copy(ei.at[pl.ds(ebase, EPT)], sall)
    pltpu.sync_copy(ei.at[pl.ds(E + ebase, EPT)], dall)

    for c in range(2):
        @pl.when(cid == c)
        def _core():
            for t in range(2):
                P = c * 2 + t
                pbase = P * PSL

                def _chunk(ch, _):
                    e0 = ch * C

                    def _idx(j, _):
                        sv = sall[pl.ds(e0 + j * 16, 16)]
                        dv = dall[pl.ds(e0 + j * 16, 16)]
                        iqb[pl.ds(j * 16, 16)] = dv * 12 + P
                        ikb[pl.ds(j * 16, 16)] = sv * 12 + (4 + P)
                        return 0

                    lax.fori_loop(0, C // 16, _idx, 0)
                    pltpu.sync_copy(table.at[iqb], qrows)
                    pltpu.sync_copy(table.at[ikb], krows)

                    def _grp(g, _):
                        qs = qrows.at[pl.ds(g * 16, 16)]
                        ks = krows.at[pl.ds(g * 16, 16)]
                        pb = g * 512
                        for e in range(16):
                            accA = (qs[e, pl.ds(0, 16)]
                                    * ks[e, pl.ds(0, 16)])
                            accB = (qs[e, pl.ds(64, 16)]
                                    * ks[e, pl.ds(64, 16)])
                            for d in range(1, DK // 16):
                                accA = accA + (qs[e, pl.ds(d * 16, 16)]
                                               * ks[e, pl.ds(d * 16, 16)])
                                accB = accB + (
                                    qs[e, pl.ds(64 + d * 16, 16)]
                                    * ks[e, pl.ds(64 + d * 16, 16)])
                            prodb[pl.ds(pb + e * 32, 16)] = accA
                            prodb[pl.ds(pb + e * 32 + 16, 16)] = accB
                        return 0

                    lax.fori_loop(0, C // 16, _grp, 0)
                    pltpu.sync_copy(
                        prodb,
                        slab.at[pl.ds(pbase + (ebase + e0) * 32, C * 32)])
                    return 0

                lax.fori_loop(0, CHUNKS, _chunk, 0)


def _softmax_kernel(p_ref, s_ref, o_ref):
    S = s_ref[...]
    sums = jnp.dot(p_ref[...], S, preferred_element_type=jnp.float32)
    ex = jnp.exp(sums * 0.125)
    o_ref[...] = jnp.dot(ex, S.T, preferred_element_type=jnp.float32)


def _edge_softmax(prods):
    S = (jnp.arange(128)[:, None] // 16 == jnp.arange(8)[None, :]
         ).astype(jnp.float32)
    NR = 4 * PSL // 128
    BE = NR // 16
    out = pl.pallas_call(
        _softmax_kernel,
        grid=(16,),
        in_specs=[
            pl.BlockSpec((BE, 128), lambda i: (i, 0)),
            pl.BlockSpec((128, 8), lambda i: (0, 0)),
        ],
        out_specs=pl.BlockSpec((BE, 128), lambda i: (i, 0)),
        out_shape=jax.ShapeDtypeStruct((NR, 128), jnp.float32),
    )(prods.reshape(NR, 128), S)
    return out.reshape(4 * PSL)


@functools.partial(
    pl.kernel,
    out_type=jax.ShapeDtypeStruct((6, NACC_OUT, RW), jnp.float32),
    mesh=_SC_MESH,
    scratch_types=[
        pltpu.VMEM((EPT,), jnp.int32),
        pltpu.VMEM((EPT,), jnp.int32),
        pltpu.VMEM((C,), jnp.int32),
        pltpu.VMEM((C,), jnp.int32),
        pltpu.VMEM((C, RW), jnp.float32),
        pltpu.VMEM((C * 32,), jnp.float32),
        pltpu.VMEM((C * 32,), jnp.float32),
        pltpu.VMEM((C, RW), jnp.float32),
        pltpu.VMEM((C, RW), jnp.float32),
        pltpu.VMEM_SHARED((NR_ACC, RW), jnp.float32),
    ],
)
def _aggregate_kernel(table, ei, exall, out,
                      sall, dall, ivb, dsm, vrows, exa, exb, msg, zblk, nsp):
    cid = lax.axis_index("c")
    sid = lax.axis_index("s")
    ebase = sid * EPT
    rbase = sid * ROWS_PT

    pltpu.sync_copy(ei.at[pl.ds(ebase, EPT)], sall)
    pltpu.sync_copy(ei.at[pl.ds(E + ebase, EPT)], dall)

    def _zrow(g, _):
        zs = zblk.at[pl.ds(g * 16, 16)]
        z = jnp.zeros((16,), jnp.float32)
        for e in range(16):
            for d in range(RW // 16):
                zs[e, pl.ds(d * 16, 16)] = z
        return 0

    lax.fori_loop(0, C // 16, _zrow, 0)

    def _zero_accum():
        for i in range(ROWS_PT // C):
            pltpu.sync_copy(zblk, nsp.at[pl.ds(rbase + i * C, C)])
        rem = ROWS_PT % C
        if rem:
            pltpu.sync_copy(
                zblk.at[pl.ds(0, rem)],
                nsp.at[pl.ds(rbase + (ROWS_PT // C) * C, rem)])

    def _drain(plane, half):
        pltpu.sync_copy(nsp.at[pl.ds(rbase, ROWS_PT)],
                        out.at[plane, pl.ds(half * NHALF + rbase, ROWS_PT)])

    for c in range(2):
        @pl.when(cid == c)
        def _core():
            for t in range(2):
                P = c * 2 + t
                pbase = P * PSL

                def _make_chunk_n(half):
                    def _chunk_n(ch, _):
                        return _chunk_n_body(ch, half)
                    return _chunk_n

                def _chunk_n_body(ch, half):
                    e0 = ch * C

                    def _idx(j, _):
                        sv = sall[pl.ds(e0 + j * 16, 16)]
                        ivb[pl.ds(j * 16, 16)] = sv * 12 + (8 + P)
                        return 0

                    lax.fori_loop(0, C // 16, _idx, 0)

                    def _sidx(j, _):
                        dv = dall[pl.ds(e0 + j * 16, 16)]
                        if half == 0:
                            dsm[pl.ds(j * 16, 16)] = jnp.minimum(dv, NHALF)
                        else:
                            tt = dv - NHALF
                            ind = jnp.minimum(jnp.maximum(0 - tt, 0), 1)
                            dsm[pl.ds(j * 16, 16)] = (
                                tt + ind * ((N - NHALF) - tt))
                        return 0

                    lax.fori_loop(0, C // 16, _sidx, 0)
                    pltpu.sync_copy(table.at[ivb], vrows)
                    pltpu.sync_copy(
                        exall.at[pl.ds(pbase + (ebase + e0) * 32, C * 32)],
                        exa)

                    def _grp(g, _):
                        vs = vrows.at[pl.ds(g * 16, 16)]
                        ms = msg.at[pl.ds(g * 16, 16)]
                        pb = g * 512
                        for e in range(16):
                            eA = exa[pl.ds(pb + e * 32, 16)]
                            eB = exa[pl.ds(pb + e * 32 + 16, 16)]
                            for d in range(DV // 16):
                                ms[e, pl.ds(d * 16, 16)] = (
                                    eA * vs[e, pl.ds(d * 16, 16)])
                                ms[e, pl.ds(64 + d * 16, 16)] = (
                                    eB * vs[e, pl.ds(64 + d * 16, 16)])
                        return 0

                    lax.fori_loop(0, C // 16, _grp, 0)
                    pltpu.sync_copy(msg, nsp.at[dsm], add=True)
                    return 0

                for half in range(2):
                    _zero_accum()
                    plsc.subcore_barrier()
                    lax.fori_loop(0, CHUNKS, _make_chunk_n(half), 0)
                    plsc.subcore_barrier()
                    _drain(P, half)
                    plsc.subcore_barrier()

            pbA = (c * 2) * PSL
            pbB = (c * 2 + 1) * PSL

            def _zpad(g, _):
                ms = msg.at[pl.ds(g * 16, 16)]
                z = jnp.zeros((16,), jnp.float32)
                for e in range(16):
                    for d in range(4, RW // 16):
                        ms[e, pl.ds(d * 16, 16)] = z
                return 0

            def _make_chunk_d(half):
                def _chunk_d(ch, _):
                    return _chunk_d_body(ch, half)
                return _chunk_d

            def _chunk_d_body(ch, half):
                e0 = ch * C

                def _sidx(j, _):
                    dv = dall[pl.ds(e0 + j * 16, 16)]
                    if half == 0:
                        dsm[pl.ds(j * 16, 16)] = jnp.minimum(dv, NHALF)
                    else:
                        tt = dv - NHALF
                        ind = jnp.minimum(jnp.maximum(0 - tt, 0), 1)
                        dsm[pl.ds(j * 16, 16)] = (
                            tt + ind * ((N - NHALF) - tt))
                    return 0

                lax.fori_loop(0, C // 16, _sidx, 0)
                pltpu.sync_copy(
                    exall.at[pl.ds(pbA + (ebase + e0) * 32, C * 32)], exa)
                pltpu.sync_copy(
                    exall.at[pl.ds(pbB + (ebase + e0) * 32, C * 32)], exb)

                def _grp(g, _):
                    ms = msg.at[pl.ds(g * 16, 16)]
                    pb = g * 512
                    for e in range(16):
                        ms[e, pl.ds(0, 16)] = exa[pl.ds(pb + e * 32, 16)]
                        ms[e, pl.ds(16, 16)] = exa[pl.ds(pb + e * 32 + 16, 16)]
                        ms[e, pl.ds(32, 16)] = exb[pl.ds(pb + e * 32, 16)]
                        ms[e, pl.ds(48, 16)] = exb[pl.ds(pb + e * 32 + 16, 16)]
                    return 0

                lax.fori_loop(0, C // 16, _grp, 0)
                pltpu.sync_copy(msg, nsp.at[dsm], add=True)
                return 0

            for half in range(2):
                _zero_accum()
                lax.fori_loop(0, C // 16, _zpad, 0)
                plsc.subcore_barrier()
                lax.fori_loop(0, CHUNKS, _make_chunk_d(half), 0)
                plsc.subcore_barrier()
                _drain(4 + c, half)
                plsc.subcore_barrier()


def _out_proj_kernel(n_ref, w_ref, b_ref, o_ref):
    xs = []
    for h in range(HEADS):
        nm = n_ref[h // 2, :, 64 * (h % 2):64 * (h % 2) + DV]
        dn = n_ref[4 + h // 4, :, 16 * (h % 4):16 * (h % 4) + 1]
        dn = jnp.where(dn == 0.0, 1.0, dn)
        xs.append(nm / dn)
    X = jnp.concatenate(xs, axis=1)
    o_ref[...] = (
        jnp.dot(X, w_ref[...], preferred_element_type=jnp.float32)
        + b_ref[...]
    )


def _out_proj(numerh, W_ffT, b_ff):
    return pl.pallas_call(
        _out_proj_kernel,
        grid=(N // BN,),
        in_specs=[
            pl.BlockSpec((6, BN, RW), lambda i: (0, i, 0)),
            pl.BlockSpec((HEADS * DV, D_EMB), lambda i: (0, 0)),
            pl.BlockSpec((1, D_EMB), lambda i: (0, 0)),
        ],
        out_specs=pl.BlockSpec((BN, D_EMB), lambda i: (i, 0)),
        out_shape=jax.ShapeDtypeStruct((N, D_EMB), jnp.float32),
    )(numerh, W_ffT, b_ff.reshape(1, D_EMB))


def kernel(x, edge_index, W_qkv, b_qkv, W_ff, b_ff):
    ei = edge_index.reshape(2 * E)
    qkv = _project(x, W_qkv.T, b_qkv)
    table = qkv.reshape(N * 12, 128)
    prods = _score_kernel(table, ei)
    exw = _edge_softmax(prods)
    acc = _aggregate_kernel(table, ei, exw)
    return _out_proj(acc, W_ff.T, b_ff)


# speedup vs baseline: 9.0013x; 1.3565x over previous
import functools

import jax
import jax.numpy as jnp
from jax import lax
from jax.experimental import pallas as pl
from jax.experimental.pallas import tpu as pltpu
from jax.experimental.pallas import tpu_sc as plsc

N = 10000
E = 160000
D_EMB = 256
HEADS = 8
DK = 64
DV = 64

BN = 1000
C = 80
SUP = 5
CS = C * SUP
TILES = 16
EPT = E // TILES
SUPERS = EPT // CS
RW = 128
PSL = E * 32
NHALF = 5120
NR_ACC = 5248
ROWS_PT = NR_ACC // TILES
NACC_OUT = NHALF + NR_ACC


def _matmul_bias_kernel(x_ref, w_ref, b_ref, o_ref):
    o_ref[...] = (
        jnp.dot(x_ref[...], w_ref[...], preferred_element_type=jnp.float32)
        + b_ref[...]
    )


def _project(x, W_T, b):
    n, d_in = x.shape
    d_out = W_T.shape[1]
    return pl.pallas_call(
        _matmul_bias_kernel,
        grid=(n // BN,),
        in_specs=[
            pl.BlockSpec((BN, d_in), lambda i: (i, 0)),
            pl.BlockSpec((d_in, d_out), lambda i: (0, 0)),
            pl.BlockSpec((1, d_out), lambda i: (0, 0)),
        ],
        out_specs=pl.BlockSpec((BN, d_out), lambda i: (i, 0)),
        out_shape=jax.ShapeDtypeStruct((n, d_out), jnp.float32),
    )(x, W_T, b.reshape(1, d_out))


_SC_MESH = plsc.VectorSubcoreMesh(core_axis_name="c", subcore_axis_name="s")


@functools.partial(
    pl.kernel,
    out_type=jax.ShapeDtypeStruct((4 * PSL,), jnp.float32),
    mesh=_SC_MESH,
    scratch_types=[
        pltpu.VMEM((CS,), jnp.int32),
        pltpu.VMEM((CS,), jnp.int32),
        [pltpu.VMEM((C,), jnp.int32) for _ in range(SUP)],
        [pltpu.VMEM((C,), jnp.int32) for _ in range(SUP)],
        pltpu.VMEM((CS, RW), jnp.float32),
        pltpu.VMEM((CS, RW), jnp.float32),
        pltpu.VMEM((CS * 32,), jnp.float32),
        pltpu.SemaphoreType.DMA,
    ],
)
def _score_kernel(table, ei, slab, svb, dvb, iqb, ikb, qrows, krows, prodb,
                  sem):
    cid = lax.axis_index("c")
    sid = lax.axis_index("s")
    ebase = sid * EPT

    for c in range(2):
        @pl.when(cid == c)
        def _core():
            for t in range(2):
                P = c * 2 + t
                pbase = P * PSL

                def _chunk(ch, _):
                    e0 = ch * CS
                    pltpu.sync_copy(ei.at[pl.ds(ebase + e0, CS)], svb)
                    pltpu.sync_copy(ei.at[pl.ds(E + ebase + e0, CS)], dvb)
                    for k in range(SUP):
                        for j in range(C // 16):
                            o = k * C + j * 16
                            sv = svb[pl.ds(o, 16)]
                            dv = dvb[pl.ds(o, 16)]
                            iqb[k][pl.ds(j * 16, 16)] = dv * 12 + P
                            ikb[k][pl.ds(j * 16, 16)] = sv * 12 + (4 + P)
                    hs = []
                    for k in range(SUP):
                        hs.append(pltpu.async_copy(
                            table.at[iqb[k]],
                            qrows.at[pl.ds(k * C, C)], sem))
                        hs.append(pltpu.async_copy(
                            table.at[ikb[k]],
                            krows.at[pl.ds(k * C, C)], sem))
                    for h in hs:
                        h.wait()

                    def _grp(g, _):
                        qs = qrows.at[pl.ds(g * 16, 16)]
                        ks = krows.at[pl.ds(g * 16, 16)]
                        pb = g * 512
                        for e in range(16):
                            accA = (qs[e, pl.ds(0, 16)]
                                    * ks[e, pl.ds(0, 16)])
                            accB = (qs[e, pl.ds(64, 16)]
                                    * ks[e, pl.ds(64, 16)])
                            for d in range(1, DK // 16):
                                accA = accA + (qs[e, pl.ds(d * 16, 16)]
                                               * ks[e, pl.ds(d * 16, 16)])
                                accB = accB + (
                                    qs[e, pl.ds(64 + d * 16, 16)]
                                    * ks[e, pl.ds(64 + d * 16, 16)])
                            prodb[pl.ds(pb + e * 32, 16)] = accA
                            prodb[pl.ds(pb + e * 32 + 16, 16)] = accB
                        return 0

                    lax.fori_loop(0, CS // 16, _grp, 0)
                    pltpu.sync_copy(
                        prodb,
                        slab.at[pl.ds(pbase + (ebase + e0) * 32, CS * 32)])
                    return 0

                lax.fori_loop(0, SUPERS, _chunk, 0)


def _softmax_kernel(p_ref, s_ref, o_ref):
    S = s_ref[...]
    sums = jnp.dot(p_ref[...], S, preferred_element_type=jnp.float32)
    ex = jnp.exp(sums * 0.125)
    o_ref[...] = jnp.dot(ex, S.T, preferred_element_type=jnp.float32)


def _edge_softmax(prods):
    S = (jnp.arange(128)[:, None] // 16 == jnp.arange(8)[None, :]
         ).astype(jnp.float32)
    NR = 4 * PSL // 128
    BE = NR // 16
    out = pl.pallas_call(
        _softmax_kernel,
        grid=(16,),
        in_specs=[
            pl.BlockSpec((BE, 128), lambda i: (i, 0)),
            pl.BlockSpec((128, 8), lambda i: (0, 0)),
        ],
        out_specs=pl.BlockSpec((BE, 128), lambda i: (i, 0)),
        out_shape=jax.ShapeDtypeStruct((NR, 128), jnp.float32),
    )(prods.reshape(NR, 128), S)
    return out.reshape(4 * PSL)


@functools.partial(
    pl.kernel,
    out_type=jax.ShapeDtypeStruct((6, NACC_OUT, RW), jnp.float32),
    mesh=_SC_MESH,
    scratch_types=[
        pltpu.VMEM((CS,), jnp.int32),
        pltpu.VMEM((CS,), jnp.int32),
        [pltpu.VMEM((C,), jnp.int32) for _ in range(SUP)],
        [pltpu.VMEM((C,), jnp.int32) for _ in range(SUP)],
        pltpu.VMEM((CS * 32,), jnp.float32),
        pltpu.VMEM((CS * 32,), jnp.float32),
        pltpu.VMEM((CS, RW), jnp.float32),
        pltpu.SemaphoreType.DMA,
        pltpu.VMEM_SHARED((NR_ACC, RW), jnp.float32),
    ],
)
def _aggregate_kernel(table, ei, exall, out,
                      svb, dvb, ivb, dsm, exa, exb, msg, sem, nsp):
    cid = lax.axis_index("c")
    sid = lax.axis_index("s")
    ebase = sid * EPT
    rbase = sid * ROWS_PT

    def _zmsg(g, _):
        ms = msg.at[pl.ds(g * 16, 16)]
        z = jnp.zeros((16,), jnp.float32)
        for e in range(16):
            for d in range(RW // 16):
                ms[e, pl.ds(d * 16, 16)] = z
        return 0

    def _zero_accum():
        pltpu.sync_copy(msg.at[pl.ds(0, ROWS_PT)],
                        nsp.at[pl.ds(rbase, ROWS_PT)])

    def _drain(plane, half):
        pltpu.sync_copy(nsp.at[pl.ds(rbase, ROWS_PT)],
                        out.at[plane, pl.ds(half * NHALF + rbase, ROWS_PT)])

    def _scatter_idx(half, dv):
        if half == 0:
            return jnp.minimum(dv, NHALF)
        tt = dv - NHALF
        ind = jnp.minimum(jnp.maximum(0 - tt, 0), 1)
        return tt + ind * ((N - NHALF) - tt)

    for c in range(2):
        @pl.when(cid == c)
        def _core():
            for t in range(2):
                P = c * 2 + t
                pbase = P * PSL

                def _make_chunk_n(half):
                    def _chunk_n(ch, _):
                        return _chunk_n_body(ch, half)
                    return _chunk_n

                def _chunk_n_body(ch, half):
                    e0 = ch * CS
                    pltpu.sync_copy(ei.at[pl.ds(ebase + e0, CS)], svb)
                    pltpu.sync_copy(ei.at[pl.ds(E + ebase + e0, CS)], dvb)
                    for k in range(SUP):
                        for j in range(C // 16):
                            o = k * C + j * 16
                            sv = svb[pl.ds(o, 16)]
                            dv = dvb[pl.ds(o, 16)]
                            ivb[k][pl.ds(j * 16, 16)] = sv * 12 + (8 + P)
                            dsm[k][pl.ds(j * 16, 16)] = _scatter_idx(half, dv)
                    hs = []
                    for k in range(SUP):
                        hs.append(pltpu.async_copy(
                            table.at[ivb[k]],
                            msg.at[pl.ds(k * C, C)], sem))
                    pltpu.sync_copy(
                        exall.at[pl.ds(pbase + (ebase + e0) * 32, CS * 32)],
                        exa)
                    for h in hs:
                        h.wait()

                    def _grp(g, _):
                        ms = msg.at[pl.ds(g * 16, 16)]
                        pb = g * 512
                        for e in range(16):
                            eA = exa[pl.ds(pb + e * 32, 16)]
                            eB = exa[pl.ds(pb + e * 32 + 16, 16)]
                            for d in range(DV // 16):
                                ms[e, pl.ds(d * 16, 16)] = (
                                    eA * ms[e, pl.ds(d * 16, 16)])
                                ms[e, pl.ds(64 + d * 16, 16)] = (
                                    eB * ms[e, pl.ds(64 + d * 16, 16)])
                        return 0

                    lax.fori_loop(0, CS // 16, _grp, 0)
                    for k in range(SUP):
                        pltpu.sync_copy(msg.at[pl.ds(k * C, C)],
                                        nsp.at[dsm[k]], add=True)
                    return 0

                for half in range(2):
                    lax.fori_loop(0, CS // 16, _zmsg, 0)
                    _zero_accum()
                    plsc.subcore_barrier()
                    lax.fori_loop(0, SUPERS, _make_chunk_n(half), 0)
                    plsc.subcore_barrier()
                    _drain(P, half)
                    plsc.subcore_barrier()

            pbA = (c * 2) * PSL
            pbB = (c * 2 + 1) * PSL

            def _make_chunk_d(half):
                def _chunk_d(ch, _):
                    return _chunk_d_body(ch, half)
                return _chunk_d

            def _chunk_d_body(ch, half):
                e0 = ch * CS
                pltpu.sync_copy(ei.at[pl.ds(E + ebase + e0, CS)], dvb)
                for k in range(SUP):
                    for j in range(C // 16):
                        o = k * C + j * 16
                        dv = dvb[pl.ds(o, 16)]
                        dsm[k][pl.ds(j * 16, 16)] = _scatter_idx(half, dv)
                ha = pltpu.async_copy(
                    exall.at[pl.ds(pbA + (ebase + e0) * 32, CS * 32)],
                    exa, sem)
                hb = pltpu.async_copy(
                    exall.at[pl.ds(pbB + (ebase + e0) * 32, CS * 32)],
                    exb, sem)
                ha.wait()
                hb.wait()

                def _grp(g, _):
                    ms = msg.at[pl.ds(g * 16, 16)]
                    pb = g * 512
                    for e in range(16):
                        ms[e, pl.ds(0, 16)] = exa[pl.ds(pb + e * 32, 16)]
                        ms[e, pl.ds(16, 16)] = exa[pl.ds(pb + e * 32 + 16, 16)]
                        ms[e, pl.ds(32, 16)] = exb[pl.ds(pb + e * 32, 16)]
                        ms[e, pl.ds(48, 16)] = exb[pl.ds(pb + e * 32 + 16, 16)]
                    return 0

                lax.fori_loop(0, CS // 16, _grp, 0)
                for k in range(SUP):
                    pltpu.sync_copy(msg.at[pl.ds(k * C, C)],
                                    nsp.at[dsm[k]], add=True)
                return 0

            for half in range(2):
                lax.fori_loop(0, CS // 16, _zmsg, 0)
                _zero_accum()
                plsc.subcore_barrier()
                lax.fori_loop(0, SUPERS, _make_chunk_d(half), 0)
                plsc.subcore_barrier()
                _drain(4 + c, half)
                plsc.subcore_barrier()


def _out_proj_kernel(n_ref, w_ref, b_ref, o_ref):
    xs = []
    for h in range(HEADS):
        nm = n_ref[h // 2, :, 64 * (h % 2):64 * (h % 2) + DV]
        dn = n_ref[4 + h // 4, :, 16 * (h % 4):16 * (h % 4) + 1]
        dn = jnp.where(dn == 0.0, 1.0, dn)
        xs.append(nm / dn)
    X = jnp.concatenate(xs, axis=1)
    o_ref[...] = (
        jnp.dot(X, w_ref[...], preferred_element_type=jnp.float32)
        + b_ref[...]
    )


def _out_proj(numerh, W_ffT, b_ff):
    return pl.pallas_call(
        _out_proj_kernel,
        grid=(N // BN,),
        in_specs=[
            pl.BlockSpec((6, BN, RW), lambda i: (0, i, 0)),
            pl.BlockSpec((HEADS * DV, D_EMB), lambda i: (0, 0)),
            pl.BlockSpec((1, D_EMB), lambda i: (0, 0)),
        ],
        out_specs=pl.BlockSpec((BN, D_EMB), lambda i: (i, 0)),
        out_shape=jax.ShapeDtypeStruct((N, D_EMB), jnp.float32),
    )(numerh, W_ffT, b_ff.reshape(1, D_EMB))


def kernel(x, edge_index, W_qkv, b_qkv, W_ff, b_ff):
    ei = edge_index.reshape(2 * E)
    qkv = _project(x, W_qkv.T, b_qkv)
    table = qkv.reshape(N * 12, 128)
    prods = _score_kernel(table, ei)
    exw = _edge_softmax(prods)
    acc = _aggregate_kernel(table, ei, exw)
    return _out_proj(acc, W_ff.T, b_ff)


# speedup vs baseline: 9.3243x; 1.0359x over previous
import functools

import jax
import jax.numpy as jnp
from jax import lax
from jax.experimental import pallas as pl
from jax.experimental.pallas import tpu as pltpu
from jax.experimental.pallas import tpu_sc as plsc

N = 10000
E = 160000
D_EMB = 256
HEADS = 8
DK = 64
DV = 64

BN = 1000
C = 80
SUP = 5
CS = C * SUP
TILES = 16
EPT = E // TILES
SUPERS = EPT // CS
RW = 128
PSL = E * 32
NHALF = 5120
NR_ACC = 5248
ROWS_PT = NR_ACC // TILES
NACC_OUT = NHALF + NR_ACC


def _matmul_bias_kernel(x_ref, w_ref, b_ref, o_ref):
    o_ref[...] = (
        jnp.dot(x_ref[...], w_ref[...], preferred_element_type=jnp.float32)
        + b_ref[...]
    )


def _project(x, W_T, b):
    n, d_in = x.shape
    d_out = W_T.shape[1]
    return pl.pallas_call(
        _matmul_bias_kernel,
        grid=(n // BN,),
        in_specs=[
            pl.BlockSpec((BN, d_in), lambda i: (i, 0)),
            pl.BlockSpec((d_in, d_out), lambda i: (0, 0)),
            pl.BlockSpec((1, d_out), lambda i: (0, 0)),
        ],
        out_specs=pl.BlockSpec((BN, d_out), lambda i: (i, 0)),
        out_shape=jax.ShapeDtypeStruct((n, d_out), jnp.float32),
    )(x, W_T, b.reshape(1, d_out))


_SC_MESH = plsc.VectorSubcoreMesh(core_axis_name="c", subcore_axis_name="s")


@functools.partial(
    pl.kernel,
    out_type=jax.ShapeDtypeStruct((4 * PSL,), jnp.float32),
    mesh=_SC_MESH,
    scratch_types=[
        pltpu.VMEM((CS,), jnp.int32),
        pltpu.VMEM((CS,), jnp.int32),
        [pltpu.VMEM((C,), jnp.int32) for _ in range(SUP)],
        [pltpu.VMEM((C,), jnp.int32) for _ in range(SUP)],
        pltpu.VMEM((CS, RW), jnp.float32),
        pltpu.VMEM((CS, RW), jnp.float32),
        pltpu.VMEM((CS * 32,), jnp.float32),
        pltpu.SemaphoreType.DMA,
    ],
)
def _score_kernel(table, ei, slab, svb, dvb, iqb, ikb, qrows, krows, prodb,
                  sem):
    cid = lax.axis_index("c")
    sid = lax.axis_index("s")
    ebase = sid * EPT

    for c in range(2):
        @pl.when(cid == c)
        def _core():
            for t in range(2):
                P = c * 2 + t
                pbase = P * PSL

                def _chunk(ch, _):
                    e0 = ch * CS
                    hsv = pltpu.async_copy(
                        ei.at[pl.ds(ebase + e0, CS)], svb, sem)
                    hdv = pltpu.async_copy(
                        ei.at[pl.ds(E + ebase + e0, CS)], dvb, sem)
                    hsv.wait()
                    hdv.wait()
                    for k in range(SUP):
                        for j in range(C // 16):
                            o = k * C + j * 16
                            sv = svb[pl.ds(o, 16)]
                            dv = dvb[pl.ds(o, 16)]
                            iqb[k][pl.ds(j * 16, 16)] = dv * 12 + P
                            ikb[k][pl.ds(j * 16, 16)] = sv * 12 + (4 + P)
                    hs = []
                    for k in range(SUP):
                        hs.append(pltpu.async_copy(
                            table.at[iqb[k]],
                            qrows.at[pl.ds(k * C, C)], sem))
                        hs.append(pltpu.async_copy(
                            table.at[ikb[k]],
                            krows.at[pl.ds(k * C, C)], sem))
                    for h in hs:
                        h.wait()

                    def _grp(g, _):
                        qs = qrows.at[pl.ds(g * 16, 16)]
                        ks = krows.at[pl.ds(g * 16, 16)]
                        pb = g * 512
                        for e in range(16):
                            accA = (qs[e, pl.ds(0, 16)]
                                    * ks[e, pl.ds(0, 16)])
                            accB = (qs[e, pl.ds(64, 16)]
                                    * ks[e, pl.ds(64, 16)])
                            for d in range(1, DK // 16):
                                accA = accA + (qs[e, pl.ds(d * 16, 16)]
                                               * ks[e, pl.ds(d * 16, 16)])
                                accB = accB + (
                                    qs[e, pl.ds(64 + d * 16, 16)]
                                    * ks[e, pl.ds(64 + d * 16, 16)])
                            prodb[pl.ds(pb + e * 32, 16)] = accA
                            prodb[pl.ds(pb + e * 32 + 16, 16)] = accB
                        return 0

                    lax.fori_loop(0, CS // 16, _grp, 0)
                    pltpu.sync_copy(
                        prodb,
                        slab.at[pl.ds(pbase + (ebase + e0) * 32, CS * 32)])
                    return 0

                lax.fori_loop(0, SUPERS, _chunk, 0)


def _softmax_kernel(p_ref, s_ref, o_ref):
    S = s_ref[...]
    sums = jnp.dot(p_ref[...], S, preferred_element_type=jnp.float32)
    ex = jnp.exp(sums * 0.125)
    o_ref[...] = jnp.dot(ex, S.T, preferred_element_type=jnp.float32)


def _edge_softmax(prods):
    S = (jnp.arange(128)[:, None] // 16 == jnp.arange(8)[None, :]
         ).astype(jnp.float32)
    NR = 4 * PSL // 128
    BE = NR // 16
    out = pl.pallas_call(
        _softmax_kernel,
        grid=(16,),
        in_specs=[
            pl.BlockSpec((BE, 128), lambda i: (i, 0)),
            pl.BlockSpec((128, 8), lambda i: (0, 0)),
        ],
        out_specs=pl.BlockSpec((BE, 128), lambda i: (i, 0)),
        out_shape=jax.ShapeDtypeStruct((NR, 128), jnp.float32),
    )(prods.reshape(NR, 128), S)
    return out.reshape(4 * PSL)


@functools.partial(
    pl.kernel,
    out_type=jax.ShapeDtypeStruct((6, NACC_OUT, RW), jnp.float32),
    mesh=_SC_MESH,
    scratch_types=[
        pltpu.VMEM((CS,), jnp.int32),
        pltpu.VMEM((CS,), jnp.int32),
        [pltpu.VMEM((C,), jnp.int32) for _ in range(SUP)],
        [pltpu.VMEM((C,), jnp.int32) for _ in range(SUP)],
        pltpu.VMEM((CS * 32,), jnp.float32),
        pltpu.VMEM((CS * 32,), jnp.float32),
        pltpu.VMEM((CS, RW), jnp.float32),
        pltpu.SemaphoreType.DMA,
        pltpu.VMEM_SHARED((NR_ACC, RW), jnp.float32),
    ],
)
def _aggregate_kernel(table, ei, exall, out,
                      svb, dvb, ivb, dsm, exa, exb, msg, sem, nsp):
    cid = lax.axis_index("c")
    sid = lax.axis_index("s")
    ebase = sid * EPT
    rbase = sid * ROWS_PT

    def _zmsg(g, _):
        ms = msg.at[pl.ds(g * 16, 16)]
        z = jnp.zeros((16,), jnp.float32)
        for e in range(16):
            for d in range(RW // 16):
                ms[e, pl.ds(d * 16, 16)] = z
        return 0

    def _zero_accum():
        pltpu.sync_copy(msg.at[pl.ds(0, ROWS_PT)],
                        nsp.at[pl.ds(rbase, ROWS_PT)])

    def _drain(plane, half):
        pltpu.sync_copy(nsp.at[pl.ds(rbase, ROWS_PT)],
                        out.at[plane, pl.ds(half * NHALF + rbase, ROWS_PT)])

    def _scatter_idx(half, dv):
        if half == 0:
            return jnp.minimum(dv, NHALF)
        tt = dv - NHALF
        ind = jnp.minimum(jnp.maximum(0 - tt, 0), 1)
        return tt + ind * ((N - NHALF) - tt)

    for c in range(2):
        @pl.when(cid == c)
        def _core():
            for t in range(2):
                P = c * 2 + t
                pbase = P * PSL

                def _make_chunk_n(half):
                    def _chunk_n(ch, _):
                        return _chunk_n_body(ch, half)
                    return _chunk_n

                def _chunk_n_body(ch, half):
                    e0 = ch * CS
                    hsv = pltpu.async_copy(
                        ei.at[pl.ds(ebase + e0, CS)], svb, sem)
                    hdv = pltpu.async_copy(
                        ei.at[pl.ds(E + ebase + e0, CS)], dvb, sem)
                    hsv.wait()
                    hdv.wait()
                    for k in range(SUP):
                        for j in range(C // 16):
                            o = k * C + j * 16
                            sv = svb[pl.ds(o, 16)]
                            dv = dvb[pl.ds(o, 16)]
                            ivb[k][pl.ds(j * 16, 16)] = sv * 12 + (8 + P)
                            dsm[k][pl.ds(j * 16, 16)] = _scatter_idx(half, dv)
                    hs = []
                    for k in range(SUP):
                        hs.append(pltpu.async_copy(
                            table.at[ivb[k]],
                            msg.at[pl.ds(k * C, C)], sem))
                    pltpu.sync_copy(
                        exall.at[pl.ds(pbase + (ebase + e0) * 32, CS * 32)],
                        exa)
                    for h in hs:
                        h.wait()

                    def _grp(g, _):
                        ms = msg.at[pl.ds(g * 16, 16)]
                        pb = g * 512
                        for e in range(16):
                            eA = exa[pl.ds(pb + e * 32, 16)]
                            eB = exa[pl.ds(pb + e * 32 + 16, 16)]
                            for d in range(DV // 16):
                                ms[e, pl.ds(d * 16, 16)] = (
                                    eA * ms[e, pl.ds(d * 16, 16)])
                                ms[e, pl.ds(64 + d * 16, 16)] = (
                                    eB * ms[e, pl.ds(64 + d * 16, 16)])
                        return 0

                    lax.fori_loop(0, CS // 16, _grp, 0)
                    hs2 = []
                    for k in range(SUP):
                        hs2.append(pltpu.async_copy(
                            msg.at[pl.ds(k * C, C)],
                            nsp.at[dsm[k]], sem, add=True))
                    for h in hs2:
                        h.wait()
                    return 0

                for half in range(2):
                    lax.fori_loop(0, CS // 16, _zmsg, 0)
                    _zero_accum()
                    plsc.subcore_barrier()
                    lax.fori_loop(0, SUPERS, _make_chunk_n(half), 0)
                    plsc.subcore_barrier()
                    _drain(P, half)
                    plsc.subcore_barrier()

            pbA = (c * 2) * PSL
            pbB = (c * 2 + 1) * PSL

            def _make_chunk_d(half):
                def _chunk_d(ch, _):
                    return _chunk_d_body(ch, half)
                return _chunk_d

            def _chunk_d_body(ch, half):
                e0 = ch * CS
                pltpu.sync_copy(ei.at[pl.ds(E + ebase + e0, CS)], dvb)
                for k in range(SUP):
                    for j in range(C // 16):
                        o = k * C + j * 16
                        dv = dvb[pl.ds(o, 16)]
                        dsm[k][pl.ds(j * 16, 16)] = _scatter_idx(half, dv)
                ha = pltpu.async_copy(
                    exall.at[pl.ds(pbA + (ebase + e0) * 32, CS * 32)],
                    exa, sem)
                hb = pltpu.async_copy(
                    exall.at[pl.ds(pbB + (ebase + e0) * 32, CS * 32)],
                    exb, sem)
                ha.wait()
                hb.wait()

                def _grp(g, _):
                    ms = msg.at[pl.ds(g * 16, 16)]
                    pb = g * 512
                    for e in range(16):
                        ms[e, pl.ds(0, 16)] = exa[pl.ds(pb + e * 32, 16)]
                        ms[e, pl.ds(16, 16)] = exa[pl.ds(pb + e * 32 + 16, 16)]
                        ms[e, pl.ds(32, 16)] = exb[pl.ds(pb + e * 32, 16)]
                        ms[e, pl.ds(48, 16)] = exb[pl.ds(pb + e * 32 + 16, 16)]
                    return 0

                lax.fori_loop(0, CS // 16, _grp, 0)
                hs2 = []
                for k in range(SUP):
                    hs2.append(pltpu.async_copy(
                        msg.at[pl.ds(k * C, C)],
                        nsp.at[dsm[k]], sem, add=True))
                for h in hs2:
                    h.wait()
                return 0

            for half in range(2):
                lax.fori_loop(0, CS // 16, _zmsg, 0)
                _zero_accum()
                plsc.subcore_barrier()
                lax.fori_loop(0, SUPERS, _make_chunk_d(half), 0)
                plsc.subcore_barrier()
                _drain(4 + c, half)
                plsc.subcore_barrier()


def _out_proj_kernel(n_ref, w_ref, b_ref, o_ref):
    xs = []
    for h in range(HEADS):
        nm = n_ref[h // 2, :, 64 * (h % 2):64 * (h % 2) + DV]
        dn = n_ref[4 + h // 4, :, 16 * (h % 4):16 * (h % 4) + 1]
        dn = jnp.where(dn == 0.0, 1.0, dn)
        xs.append(nm / dn)
    X = jnp.concatenate(xs, axis=1)
    o_ref[...] = (
        jnp.dot(X, w_ref[...], preferred_element_type=jnp.float32)
        + b_ref[...]
    )


def _out_proj(numerh, W_ffT, b_ff):
    return pl.pallas_call(
        _out_proj_kernel,
        grid=(N // BN,),
        in_specs=[
            pl.BlockSpec((6, BN, RW), lambda i: (0, i, 0)),
            pl.BlockSpec((HEADS * DV, D_EMB), lambda i: (0, 0)),
            pl.BlockSpec((1, D_EMB), lambda i: (0, 0)),
        ],
        out_specs=pl.BlockSpec((BN, D_EMB), lambda i: (i, 0)),
        out_shape=jax.ShapeDtypeStruct((N, D_EMB), jnp.float32),
    )(numerh, W_ffT, b_ff.reshape(1, D_EMB))


def kernel(x, edge_index, W_qkv, b_qkv, W_ff, b_ff):
    ei = edge_index.reshape(2 * E)
    qkv = _project(x, W_qkv.T, b_qkv)
    table = qkv.reshape(N * 12, 128)
    prods = _score_kernel(table, ei)
    exw = _edge_softmax(prods)
    acc = _aggregate_kernel(table, ei, exw)
    return _out_proj(acc, W_ff.T, b_ff)
